# baseline (speedup 1.0000x reference)
import jax
import jax.numpy as jnp
from jax import lax
from jax.experimental import pallas as pl
from jax.experimental.pallas import tpu as pltpu
from jax.experimental.pallas import tpu_sc as plsc

NODE_NUM = 1000000
N_UPDATES = 4194304

LANES = 128
ROWS = N_UPDATES // LANES
N_TILES = 16
ROWS_PER_TILE = ROWS // N_TILES
GROUP_ROWS = 32
N_GROUPS = ROWS_PER_TILE // GROUP_ROWS

HALF = NODE_NUM // 2
ACC = 512000
TRASH = HALF

TILE_OUT = 31264
TILE_OUT_LAST = HALF - 15 * TILE_OUT
CHUNK = 2048
FULL_CHUNKS = 15
TAIL0 = TILE_OUT - FULL_CHUNKS * CHUNK
TAIL15 = TILE_OUT_LAST - FULL_CHUNKS * CHUNK
ZBUF = 4000


def _body(idx_hbm, val_hbm, str_hbm, out_hbm,
          sums, cnts, idx_raw, idx2, vals, ones_r, zbuf, sv, cv, stv, ov,
          sem_in, sem_sc):
  c = lax.axis_index("c")
  t = lax.axis_index("s")

  @pl.loop(0, ZBUF // 16)
  def _(i):
    zbuf[pl.ds(i * 16, 16)] = jnp.zeros((16,), jnp.float32)

  @pl.loop(0, GROUP_ROWS)
  def _(j):
    row = ones_r.at[j]
    for b in range(LANES // 16):
      row[pl.ds(b * 16, 16)] = jnp.ones((16,), jnp.float32)

  per_tile = ACC // N_TILES

  @pl.loop(0, per_tile // ZBUF)
  def _(i):
    off = t * per_tile + i * ZBUF
    pltpu.sync_copy(zbuf, sums.at[pl.ds(off, ZBUF)])
    pltpu.sync_copy(zbuf, cnts.at[pl.ds(off, ZBUF)])

  plsc.subcore_barrier()

  base = c * HALF
  LAG = 8

  def sc_row_descs(j):
    return (
        pltpu.make_async_copy(vals.at[j], sums.at[idx2.at[j]], sem_sc),
        pltpu.make_async_copy(ones_r.at[j], cnts.at[idx2.at[j]], sem_sc),
    )

  @pl.loop(0, N_GROUPS)
  def _(g):
    row = t * ROWS_PER_TILE + g * GROUP_ROWS
    pltpu.sync_copy(idx_hbm.at[pl.ds(row, GROUP_ROWS)], idx_raw)
    pltpu.sync_copy(val_hbm.at[pl.ds(row, GROUP_ROWS)], vals)

    @pl.loop(0, GROUP_ROWS)
    def _(j):
      src_row = idx_raw.at[j]
      dst_row = idx2.at[j]
      for b in range(LANES // 16):
        v = src_row[pl.ds(b * 16, 16)]
        loc = v - base
        m = loc.astype(jnp.uint32) < jnp.uint32(HALF)
        dst_row[pl.ds(b * 16, 16)] = jnp.where(m, loc, TRASH)

    @pl.loop(0, GROUP_ROWS // LAG)
    def _(bi):
      descs = []
      for b in range(LAG):
        descs.extend(sc_row_descs(bi * LAG + b))
      for d in descs:
        d.start(add=True)
      for d in descs:
        d.wait()

  plsc.subcore_barrier()

  lbase = t * TILE_OUT
  abase = c * HALF + lbase

  def compute(off, size):
    pltpu.sync_copy(sums.at[pl.ds(lbase + off, size)], sv.at[pl.ds(0, size)])
    pltpu.sync_copy(cnts.at[pl.ds(lbase + off, size)], cv.at[pl.ds(0, size)])
    pltpu.sync_copy(str_hbm.at[pl.ds(abase + off, size)], stv.at[pl.ds(0, size)])
    for i in range(size // 16):
      s = sv[pl.ds(i * 16, 16)]
      n = cv[pl.ds(i * 16, 16)]
      z = stv[pl.ds(i * 16, 16)]
      ov[pl.ds(i * 16, 16)] = z + s / jnp.maximum(n, 1.0)
    pltpu.sync_copy(ov.at[pl.ds(0, size)], out_hbm.at[pl.ds(abase + off, size)])

  @pl.loop(0, FULL_CHUNKS)
  def _(g):
    compute(g * CHUNK, CHUNK)

  @pl.when(t < N_TILES - 1)
  def _():
    compute(FULL_CHUNKS * CHUNK, TAIL0)

  @pl.when(t == N_TILES - 1)
  def _():
    compute(FULL_CHUNKS * CHUNK, TAIL15)


@jax.jit
def kernel(node_strengths, node_indices, strengths):
  idx2d = node_indices.reshape(ROWS, LANES)
  val2d = node_strengths.reshape(ROWS, LANES)
  run = pl.kernel(
      _body,
      out_type=jax.ShapeDtypeStruct((NODE_NUM,), jnp.float32),
      mesh=plsc.VectorSubcoreMesh(core_axis_name="c", subcore_axis_name="s"),
      scratch_types=[
          pltpu.VMEM_SHARED((ACC,), jnp.float32),
          pltpu.VMEM_SHARED((ACC,), jnp.float32),
          pltpu.VMEM((GROUP_ROWS, LANES), jnp.int32),
          pltpu.VMEM((GROUP_ROWS, LANES), jnp.int32),
          pltpu.VMEM((GROUP_ROWS, LANES), jnp.float32),
          pltpu.VMEM((GROUP_ROWS, LANES), jnp.float32),
          pltpu.VMEM((ZBUF,), jnp.float32),
          pltpu.VMEM((CHUNK,), jnp.float32),
          pltpu.VMEM((CHUNK,), jnp.float32),
          pltpu.VMEM((CHUNK,), jnp.float32),
          pltpu.VMEM((CHUNK,), jnp.float32),
          pltpu.SemaphoreType.DMA,
          pltpu.SemaphoreType.DMA,
      ],
  )
  return run(idx2d, val2d, strengths)


# speedup vs baseline: 18.0295x; 18.0295x over previous
import jax
import jax.numpy as jnp
from jax import lax
from jax.experimental import pallas as pl
from jax.experimental.pallas import tpu as pltpu
from jax.experimental.pallas import tpu_sc as plsc

NODE_NUM = 1000000
N_UPDATES = 4194304

LANES = 128
ROWS = N_UPDATES // LANES
N_TILES = 16
ROWS_PER_TILE = ROWS // N_TILES
GROUP_ROWS = 32
N_GROUPS = ROWS_PER_TILE // GROUP_ROWS

HALF = NODE_NUM // 2
ACC = 512000
TRASH = HALF
TRASH_SPREAD = 8192

TILE_OUT = 31264
TILE_OUT_LAST = HALF - 15 * TILE_OUT
CHUNK = 2048
FULL_CHUNKS = 15
TAIL0 = TILE_OUT - FULL_CHUNKS * CHUNK
TAIL15 = TILE_OUT_LAST - FULL_CHUNKS * CHUNK
ZBUF = 4000


def _body(idx_hbm, val_hbm, str_hbm, out_hbm,
          sums, cnts, idx_raw, idx2, vals, ones_r, zbuf, sv, cv, stv, ov,
          sem_in, sem_sc):
  c = lax.axis_index("c")
  t = lax.axis_index("s")

  def in_descs(g, p):
    row = t * ROWS_PER_TILE + g * GROUP_ROWS
    return (
        pltpu.make_async_copy(idx_hbm.at[pl.ds(row, GROUP_ROWS)],
                              idx_raw.at[p], sem_in),
        pltpu.make_async_copy(val_hbm.at[pl.ds(row, GROUP_ROWS)],
                              vals.at[p], sem_in),
    )

  for d in in_descs(0, 0):
    d.start()

  @pl.loop(0, ZBUF // 16)
  def _(i):
    zbuf[pl.ds(i * 16, 16)] = jnp.zeros((16,), jnp.float32)

  @pl.loop(0, GROUP_ROWS)
  def _(j):
    row = ones_r.at[j]
    for b in range(LANES // 16):
      row[pl.ds(b * 16, 16)] = jnp.ones((16,), jnp.float32)

  per_tile = ACC // N_TILES

  @pl.loop(0, per_tile // ZBUF)
  def _(i):
    off = t * per_tile + i * ZBUF
    pltpu.sync_copy(zbuf, sums.at[pl.ds(off, ZBUF)])
    pltpu.sync_copy(zbuf, cnts.at[pl.ds(off, ZBUF)])

  plsc.subcore_barrier()

  base = c * HALF
  LAG = 8

  def sc_row_descs(p, j):
    return (
        pltpu.make_async_copy(vals.at[p].at[j],
                              sums.at[idx2.at[p].at[j]], sem_sc),
        pltpu.make_async_copy(ones_r.at[j],
                              cnts.at[idx2.at[p].at[j]], sem_sc),
    )

  @pl.loop(0, N_GROUPS // 2)
  def _(o):
    for p in range(2):
      g = o * 2 + p
      for d in in_descs(g, p):
        d.wait()

      @pl.loop(0, GROUP_ROWS)
      def _(j):
        src_row = idx_raw.at[p].at[j]
        dst_row = idx2.at[p].at[j]
        for b in range(LANES // 16):
          v = src_row[pl.ds(b * 16, 16)]
          loc = v - base
          m = loc.astype(jnp.uint32) < jnp.uint32(HALF)
          trash = TRASH + (v & (TRASH_SPREAD - 1))
          dst_row[pl.ds(b * 16, 16)] = jnp.where(m, loc, trash)

      @pl.when(g > 0)
      def _():
        @pl.loop(0, GROUP_ROWS)
        def _(j):
          for d in sc_row_descs(1 - p, j):
            d.wait()

      @pl.when(g < N_GROUPS - 1)
      def _():
        for d in in_descs(g + 1, 1 - p):
          d.start()

      @pl.loop(0, GROUP_ROWS)
      def _(j):
        for d in sc_row_descs(p, j):
          d.start(add=True)

  @pl.loop(0, GROUP_ROWS)
  def _(j):
    for d in sc_row_descs((N_GROUPS - 1) % 2, j):
      d.wait()

  plsc.subcore_barrier()

  lbase = t * TILE_OUT
  abase = c * HALF + lbase

  def compute(off, size):
    pltpu.sync_copy(sums.at[pl.ds(lbase + off, size)], sv.at[pl.ds(0, size)])
    pltpu.sync_copy(cnts.at[pl.ds(lbase + off, size)], cv.at[pl.ds(0, size)])
    pltpu.sync_copy(str_hbm.at[pl.ds(abase + off, size)], stv.at[pl.ds(0, size)])
    for i in range(size // 16):
      s = sv[pl.ds(i * 16, 16)]
      n = cv[pl.ds(i * 16, 16)]
      z = stv[pl.ds(i * 16, 16)]
      ov[pl.ds(i * 16, 16)] = z + s / jnp.maximum(n, 1.0)
    pltpu.sync_copy(ov.at[pl.ds(0, size)], out_hbm.at[pl.ds(abase + off, size)])

  @pl.loop(0, FULL_CHUNKS)
  def _(g):
    compute(g * CHUNK, CHUNK)

  @pl.when(t < N_TILES - 1)
  def _():
    compute(FULL_CHUNKS * CHUNK, TAIL0)

  @pl.when(t == N_TILES - 1)
  def _():
    compute(FULL_CHUNKS * CHUNK, TAIL15)


@jax.jit
def kernel(node_strengths, node_indices, strengths):
  idx2d = node_indices.reshape(ROWS, LANES)
  val2d = node_strengths.reshape(ROWS, LANES)
  run = pl.kernel(
      _body,
      out_type=jax.ShapeDtypeStruct((NODE_NUM,), jnp.float32),
      mesh=plsc.VectorSubcoreMesh(core_axis_name="c", subcore_axis_name="s"),
      scratch_types=[
          pltpu.VMEM_SHARED((ACC,), jnp.float32),
          pltpu.VMEM_SHARED((ACC,), jnp.float32),
          pltpu.VMEM((2, GROUP_ROWS, LANES), jnp.int32),
          pltpu.VMEM((2, GROUP_ROWS, LANES), jnp.int32),
          pltpu.VMEM((2, GROUP_ROWS, LANES), jnp.float32),
          pltpu.VMEM((GROUP_ROWS, LANES), jnp.float32),
          pltpu.VMEM((ZBUF,), jnp.float32),
          pltpu.VMEM((CHUNK,), jnp.float32),
          pltpu.VMEM((CHUNK,), jnp.float32),
          pltpu.VMEM((CHUNK,), jnp.float32),
          pltpu.VMEM((CHUNK,), jnp.float32),
          pltpu.SemaphoreType.DMA,
          pltpu.SemaphoreType.DMA,
      ],
  )
  return run(idx2d, val2d, strengths)
